# baseline (speedup 1.0000x reference)
import functools

import jax
import jax.numpy as jnp
from jax import lax
from jax.experimental import pallas as pl
from jax.experimental.pallas import tpu as pltpu
from jax.experimental.pallas import tpu_sc as plsc

C = 1000
D = 2048
B = 256
NROW = 1280
TROW = 1024
NW = 32
WCOL = D // NW
RCH = 128


def _entropy_cls(logits, n):
    m = jnp.max(logits, axis=1, keepdims=True)
    eu = jnp.exp(logits - m)
    s = jnp.sum(eu, axis=1, keepdims=True)
    p = eu / s
    logp = logits - m - jnp.log(s)
    ent = -jnp.sum(p * logp, axis=1)
    iota = lax.broadcasted_iota(jnp.int32, logits.shape, 1)
    cls = jnp.min(jnp.where(logits == m, iota, C), axis=1)
    return ent, cls


def _tc1_body(x_ref, wf_ref, wh_ref, z_ref, nt_ref, cls_ref):
    xv = x_ref[...]
    wf = wf_ref[...]
    wh = wh_ref[...]
    dn = (((1,), (1,)), ((), ()))
    z = lax.dot_general(xv, wf, dn, preferred_element_type=jnp.float32)
    z_ref[...] = z
    zt = lax.dot_general(wf, xv, dn, preferred_element_type=jnp.float32)
    g = lax.dot_general(wh, wh, dn, preferred_element_type=jnp.float32)
    went, wcls = _entropy_cls(g, C)
    p = lax.dot_general(z, wh, dn, preferred_element_type=jnp.float32)
    ent, ycls = _entropy_cls(p, C)
    wht = jnp.transpose(wh)

    cls = jnp.concatenate([wcls, ycls, jnp.full((NROW - C - B,), C, jnp.int32)])
    e = jnp.concatenate([went, ent, jnp.zeros((NROW - C - B,), jnp.float32)])
    cls_ref[...] = cls[None, :]

    idx = lax.broadcasted_iota(jnp.int32, (NROW, NROW), 1)
    idxT = lax.broadcasted_iota(jnp.int32, (NROW, NROW), 0)
    eqc = cls[:, None] == cls[None, :]
    later = (e[None, :] > e[:, None]) | ((e[None, :] == e[:, None]) & (idx > idxT))
    keep = jnp.any(eqc & later, axis=1)

    nrm_w = jnp.sqrt(jnp.sum(wh * wh, axis=1))
    nrm_z = jnp.sqrt(jnp.sum(z * z, axis=1))
    nrm = jnp.concatenate([nrm_w, nrm_z, jnp.ones((NROW - C - B,), jnp.float32)])
    scale = jnp.where(keep, 1.0 / jnp.maximum(nrm, 1e-12), 0.0)
    nt_ref[:, 0:C] = wht * scale[None, 0:C]
    nt_ref[:, C:C + B] = zt * scale[None, C:C + B]
    nt_ref[:, C + B:NROW] = jnp.zeros((D, NROW - C - B), jnp.float32)


_tc1 = pl.pallas_call(
    _tc1_body,
    out_shape=[
        jax.ShapeDtypeStruct((B, D), jnp.float32),
        jax.ShapeDtypeStruct((D, NROW), jnp.float32),
        jax.ShapeDtypeStruct((1, NROW), jnp.int32),
    ],
)


def _tc2_body(z_ref, o3_ref, pred_ref, prob_ref):
    dn = (((1,), (0,)), ((), ()))
    acc = jnp.zeros((B, TROW), jnp.float32)
    nrm2 = jnp.zeros((TROW,), jnp.float32)
    for w in range(NW):
        zw = z_ref[:, w * WCOL:(w + 1) * WCOL]
        m = o3_ref[w]
        acc = acc + lax.dot_general(zw, m, dn, preferred_element_type=jnp.float32)
        nrm2 = nrm2 + jnp.sum(m * m, axis=0)
    invn = 1.0 / jnp.maximum(jnp.sqrt(nrm2), 1e-12)
    pred = (acc * invn[None, :])[:, 0:C]
    pred_ref[...] = pred
    m2 = jnp.max(pred, axis=1, keepdims=True)
    s = jnp.sum(jnp.exp(pred - m2), axis=1, keepdims=True)
    prob_ref[...] = jnp.exp(pred[:, 1:2] - m2) / s


_tc2 = pl.pallas_call(
    _tc2_body,
    out_shape=[
        jax.ShapeDtypeStruct((B, C), jnp.float32),
        jax.ShapeDtypeStruct((B, 1), jnp.float32),
    ],
)


def _sc_body(nt_hbm, cls_hbm, out_hbm, idx_v, vals_v, table_f):
    cid = lax.axis_index("c")
    sid = lax.axis_index("s")
    w = sid * 2 + cid
    row = w * WCOL
    pltpu.sync_copy(cls_hbm, idx_v)
    zv = jnp.zeros((16,), jnp.float32)

    @pl.loop(0, WCOL)
    def _zrow(r):
        for j in range(TROW // 16):
            table_f[r, pl.ds(j * 16, 16)] = zv

    @pl.loop(0, NROW // RCH)
    def _chunk(k):
        base = pl.multiple_of(k * RCH, RCH)
        pltpu.sync_copy(nt_hbm.at[pl.ds(row, WCOL), pl.ds(base, RCH)], vals_v)
        for g in range(RCH // 16):
            cls16 = idx_v[pl.ds(base + g * 16, 16)]
            for c in range(WCOL):
                v = vals_v[c, pl.ds(g * 16, 16)]
                plsc.addupdate_scatter(table_f, [jnp.full((16,), c, jnp.int32), cls16], v)

    pltpu.sync_copy(table_f, out_hbm.at[w])


@functools.cache
def _sc_scatter():
    return functools.partial(
        pl.kernel,
        mesh=plsc.VectorSubcoreMesh(core_axis_name="c", subcore_axis_name="s"),
        out_type=jax.ShapeDtypeStruct((NW, WCOL, TROW), jnp.float32),
        scratch_types=[
            pltpu.VMEM((NROW,), jnp.int32),
            pltpu.VMEM((WCOL, RCH), jnp.float32),
            pltpu.VMEM((WCOL, TROW), jnp.float32),
        ],
        compiler_params=pltpu.CompilerParams(needs_layout_passes=False),
    )(_sc_body)


@jax.jit
def kernel(x, W_feat, W_head):
    z, n_t, cls2d = _tc1(x, W_feat, W_head)
    o = _sc_scatter()(n_t, cls2d.reshape(NROW))
    pred, prob = _tc2(z, o)
    return pred, prob.reshape(B), z
